# baseline (speedup 1.0000x reference)
import jax
import jax.numpy as jnp
from jax.experimental import pallas as pl
from jax.experimental.pallas import tpu as pltpu

N, E = 1_000_000, 256
BLOCK = 8_000


def _body(x_ref, w_ref, o_ref):
    x = x_ref[...]
    w = w_ref[...]
    xx = x * x
    s = (jnp.dot(x[:, :128], w[:128, :], preferred_element_type=jnp.float32)
         + jnp.dot(x[:, 128:], w[128:, :], preferred_element_type=jnp.float32))
    num = (jnp.dot(xx[:, :128], w[:128, :], preferred_element_type=jnp.float32)
           + jnp.dot(xx[:, 128:], w[128:, :], preferred_element_type=jnp.float32))
    o_ref[...] = jnp.where(s == 0.0, 0.0, num / s)


def kernel(x, w):
    grid = (N // BLOCK,)
    out = pl.pallas_call(
        _body,
        grid=grid,
        in_specs=[
            pl.BlockSpec((BLOCK, E), lambda i: (i, 0)),
            pl.BlockSpec((E, 1), lambda i: (0, 0)),
        ],
        out_specs=pl.BlockSpec((BLOCK, 1), lambda i: (i, 0)),
        out_shape=jax.ShapeDtypeStruct((N, 1), jnp.float32),
        compiler_params=pltpu.CompilerParams(
            dimension_semantics=("parallel",),
        ),
    )(x, w)
    return out.reshape(N)
